# Initial kernel scaffold; baseline (speedup 1.0000x reference)
import jax
import jax.numpy as jnp
from jax.experimental import pallas as pl


def kernel(x, edge_index, W1, b1, W2, b2):
    raise NotImplementedError("write your pallas kernel here")


# speedup vs baseline: 2.9061x; 2.9061x over previous
import jax
import jax.numpy as jnp
from jax import lax
from jax.experimental import pallas as pl
from jax.experimental.pallas import tpu as pltpu
from jax.experimental.pallas import tpu_sc as plsc

N_NODES = 10000
N_PAD = 10240
E_PAD = 163840
CHUNK = 128
MBLK = 512
NROW = N_PAD // 16
NSUB = NROW // CHUNK


def _tc1_body(x_ref, w_ref, yself_ref, qab_ref):
    y = jnp.dot(x_ref[...], w_ref[...], preferred_element_type=jnp.float32)
    yself_ref[...] = y[:, :256]
    qab_ref[0] = y[:, 256:384]
    qab_ref[1] = y[:, 384:512]


def _tc1(xp, w1h):
    nb = N_PAD // MBLK
    return pl.pallas_call(
        _tc1_body,
        grid=(nb,),
        in_specs=[
            pl.BlockSpec((MBLK, 256), lambda m: (m, 0)),
            pl.BlockSpec((256, 512), lambda m: (0, 0)),
        ],
        out_specs=[
            pl.BlockSpec((MBLK, 256), lambda m: (m, 0)),
            pl.BlockSpec((2, MBLK, 128), lambda m: (0, m, 0)),
        ],
        out_shape=[
            jax.ShapeDtypeStruct((N_PAD, 256), jnp.float32),
            jax.ShapeDtypeStruct((2, N_PAD, 128), jnp.float32),
        ],
    )(xp, w1h)


def _tc2_body(yself_ref, s1_ref, d0_ref, d1_ref, b1_ref, w2h_ref,
              zself_ref, q2_ref):
    m = pl.program_id(0)
    deg = jnp.maximum(d0_ref[...] + d1_ref[...], 1.0)
    inv = 1.0 / deg
    h_n = jnp.concatenate([s1_ref[0], s1_ref[1]], axis=1)
    pre = yself_ref[...] + b1_ref[...] + h_n * inv
    row = m * MBLK + lax.broadcasted_iota(jnp.int32, (MBLK, 1), 0)
    out1 = jnp.where(row < N_NODES, jnp.maximum(pre, 0.0), 0.0)
    z = jnp.dot(out1, w2h_ref[...], preferred_element_type=jnp.float32)
    zself_ref[...] = z[:, :128]
    q2_ref[...] = z[:, 128:]


def _tc2(yself, s1, deg8, b1, w2h):
    nb = N_PAD // MBLK
    return pl.pallas_call(
        _tc2_body,
        grid=(nb,),
        in_specs=[
            pl.BlockSpec((MBLK, 256), lambda m: (m, 0)),
            pl.BlockSpec((2, MBLK, 128), lambda m: (0, m, 0)),
            pl.BlockSpec((MBLK, 1), lambda m: (m, 0)),
            pl.BlockSpec((MBLK, 1), lambda m: (m + nb, 0)),
            pl.BlockSpec((1, 256), lambda m: (0, 0)),
            pl.BlockSpec((256, 256), lambda m: (0, 0)),
        ],
        out_specs=[
            pl.BlockSpec((MBLK, 128), lambda m: (m, 0)),
            pl.BlockSpec((MBLK, 128), lambda m: (m, 0)),
        ],
        out_shape=[
            jax.ShapeDtypeStruct((N_PAD, 128), jnp.float32),
            jax.ShapeDtypeStruct((N_PAD, 128), jnp.float32),
        ],
    )(yself, s1, deg8, deg8, b1, w2h)


def _tc3_body(zself_ref, s2a_ref, s2b_ref, d0_ref, d1_ref, b2_ref, out_ref):
    deg = jnp.maximum(d0_ref[...] + d1_ref[...], 1.0)
    inv = 1.0 / deg
    s2 = s2a_ref[...] + s2b_ref[...]
    out_ref[...] = zself_ref[...] + b2_ref[...] + s2 * inv


def _tc3(zself, s2p, deg8, b2):
    nb = N_PAD // MBLK
    return pl.pallas_call(
        _tc3_body,
        grid=(nb,),
        in_specs=[
            pl.BlockSpec((MBLK, 128), lambda m: (m, 0)),
            pl.BlockSpec((MBLK, 128), lambda m: (m, 0)),
            pl.BlockSpec((MBLK, 128), lambda m: (m + nb, 0)),
            pl.BlockSpec((MBLK, 1), lambda m: (m, 0)),
            pl.BlockSpec((MBLK, 1), lambda m: (m + nb, 0)),
            pl.BlockSpec((1, 128), lambda m: (0, 0)),
        ],
        out_specs=pl.BlockSpec((MBLK, 128), lambda m: (m, 0)),
        out_shape=jax.ShapeDtypeStruct((N_PAD, 128), jnp.float32),
    )(zself, s2p, s2p, deg8, deg8, b2)


_MESH = dict(core_axis_name="c", subcore_axis_name="s",
             num_cores=2, num_subcores=16)

def _zero_vmem(ref):
    nrow, ncol = ref.shape

    def step(i, carry):
        r = i // (ncol // 16)
        c = lax.rem(i, ncol // 16) * 16
        ref[r, pl.ds(c, 16)] = jnp.zeros((16,), jnp.float32)
        return carry

    lax.fori_loop(0, nrow * (ncol // 16), step, 0, unroll=False)


def _zero_vmem1(ref):
    def step(i, carry):
        ref[pl.ds(i * 16, 16)] = jnp.zeros((16,), jnp.float32)
        return carry

    lax.fori_loop(0, ref.shape[0] // 16, step, 0, unroll=False)


def _sc1_body(qab, src2, dstp, emask1,
              s1, degp1,
              sidx, didx, mval1, rows, acc, dacc1, sem):
    cid = lax.axis_index("c")
    sid = lax.axis_index("s")
    _zero_vmem(rows)
    _zero_vmem1(mval1)
    for j in range(NSUB):
        zs = pl.ds(sid * NROW + j * CHUNK, CHUNK)
        pltpu.sync_copy(rows, acc.at[zs])
        pltpu.sync_copy(mval1, dacc1.at[zs])
    plsc.subcore_barrier()

    epw = E_PAD // 16
    base = sid * epw
    sbase = cid * E_PAD + base

    def step(i, carry):
        pltpu.sync_copy(src2.at[pl.ds(sbase + i * CHUNK, CHUNK)], sidx)
        pltpu.sync_copy(dstp.at[pl.ds(base + i * CHUNK, CHUNK)], didx)
        pltpu.async_copy(qab.at[sidx], rows, sem).wait()
        pltpu.sync_copy(rows, acc.at[didx], add=True)
        return carry

    lax.fori_loop(0, epw // CHUNK, step, 0, unroll=False)

    dpw = E_PAD // 32
    dbase = cid * (E_PAD // 2) + sid * dpw

    def dstep(i, carry):
        off = dbase + i * CHUNK
        pltpu.sync_copy(dstp.at[pl.ds(off, CHUNK)], didx)
        pltpu.sync_copy(emask1.at[pl.ds(off, CHUNK)], mval1)
        pltpu.sync_copy(mval1, dacc1.at[didx], add=True)
        return carry

    lax.fori_loop(0, dpw // CHUNK, dstep, 0, unroll=False)
    plsc.subcore_barrier()

    for j in range(NSUB):
        zs = pl.ds(sid * NROW + j * CHUNK, CHUNK)
        hs = pl.ds(cid * N_PAD + sid * NROW + j * CHUNK, CHUNK)
        pltpu.sync_copy(acc.at[zs], rows)
        pltpu.sync_copy(rows, s1.at[hs])
        pltpu.sync_copy(dacc1.at[zs], mval1)
        pltpu.sync_copy(mval1, degp1.at[hs])


def _sc1(qab, src2, dstp, emask1):
    mesh = plsc.VectorSubcoreMesh(**_MESH)
    f = pl.kernel(
        _sc1_body,
        out_type=[
            jax.ShapeDtypeStruct((2 * N_PAD, 128), jnp.float32),
            jax.ShapeDtypeStruct((2 * N_PAD,), jnp.float32),
        ],
        mesh=mesh,
        scratch_types=[
            pltpu.VMEM((CHUNK,), jnp.int32),
            pltpu.VMEM((CHUNK,), jnp.int32),
            pltpu.VMEM((CHUNK,), jnp.float32),
            pltpu.VMEM((CHUNK, 128), jnp.float32),
            pltpu.VMEM_SHARED((N_PAD, 128), jnp.float32),
            pltpu.VMEM_SHARED((N_PAD,), jnp.float32),
            pltpu.SemaphoreType.DMA,
        ],
    )
    return f(qab, src2, dstp, emask1)


def _sc2_body(q2, srcp, dstp,
              s2p,
              sidx, didx, rows, acc, sem):
    cid = lax.axis_index("c")
    sid = lax.axis_index("s")
    _zero_vmem(rows)
    for j in range(NSUB):
        zs = pl.ds(sid * NROW + j * CHUNK, CHUNK)
        pltpu.sync_copy(rows, acc.at[zs])
    plsc.subcore_barrier()

    epw = E_PAD // 32
    base = cid * (E_PAD // 2) + sid * epw

    def step(i, carry):
        off = base + i * CHUNK
        pltpu.sync_copy(srcp.at[pl.ds(off, CHUNK)], sidx)
        pltpu.sync_copy(dstp.at[pl.ds(off, CHUNK)], didx)
        pltpu.async_copy(q2.at[sidx], rows, sem).wait()
        pltpu.sync_copy(rows, acc.at[didx], add=True)
        return carry

    lax.fori_loop(0, epw // CHUNK, step, 0, unroll=False)
    plsc.subcore_barrier()
    for j in range(NSUB):
        zs = pl.ds(sid * NROW + j * CHUNK, CHUNK)
        hs = pl.ds(cid * N_PAD + sid * NROW + j * CHUNK, CHUNK)
        pltpu.sync_copy(acc.at[zs], rows)
        pltpu.sync_copy(rows, s2p.at[hs])


def _sc2(q2, srcp, dstp):
    mesh = plsc.VectorSubcoreMesh(**_MESH)
    f = pl.kernel(
        _sc2_body,
        out_type=jax.ShapeDtypeStruct((2 * N_PAD, 128), jnp.float32),
        mesh=mesh,
        scratch_types=[
            pltpu.VMEM((CHUNK,), jnp.int32),
            pltpu.VMEM((CHUNK,), jnp.int32),
            pltpu.VMEM((CHUNK, 128), jnp.float32),
            pltpu.VMEM_SHARED((N_PAD, 128), jnp.float32),
            pltpu.SemaphoreType.DMA,
        ],
    )
    return f(q2, srcp, dstp)


def kernel(x, edge_index, W1, b1, W2, b2):
    src = edge_index[0].astype(jnp.int32)
    dst = edge_index[1].astype(jnp.int32)
    npad = E_PAD - src.shape[0]
    srcp = jnp.pad(src, (0, npad), constant_values=N_NODES)
    dstp = jnp.pad(dst, (0, npad), constant_values=0)
    src2 = jnp.concatenate([srcp, srcp + N_PAD])
    emask1 = jnp.pad(jnp.ones((src.shape[0],), jnp.float32), (0, npad))

    xp = jnp.pad(x, ((0, N_PAD - N_NODES), (0, 0)))
    w1h = jnp.concatenate([W1[:256], W1[256:]], axis=1)
    w2h = jnp.concatenate([W2[:256], W2[256:]], axis=1)

    yself, qab = _tc1(xp, w1h)
    qab2 = qab.reshape(2 * N_PAD, 128)
    s1, deg1 = _sc1(qab2, src2, dstp, emask1)
    s1 = s1.reshape(2, N_PAD, 128)
    degc = deg1.reshape(2 * N_PAD, 1)
    zself, q2 = _tc2(yself, s1, degc, b1.reshape(1, 256), w2h)
    s2p = _sc2(q2, srcp, dstp)
    out = _tc3(zself, s2p, degc, b2.reshape(1, 128))
    return out[:N_NODES]
